# Initial kernel scaffold; baseline (speedup 1.0000x reference)
import jax
import jax.numpy as jnp
from jax.experimental import pallas as pl


def kernel(x, Wr, centroids, temperature, weight, bias):
    raise NotImplementedError("write your pallas kernel here")


# speedup vs baseline: 12.1808x; 12.1808x over previous
import functools

import jax
import jax.numpy as jnp
from jax.experimental import pallas as pl

IN_FEATURES = 1024
NUM_HEADS = 16
HEAD_DIM = 128
TOP_K = 2
HIDDEN = 256


def _sparse_query_kernel(x_ref, wr_ref, c_ref, t_ref, w_ref, b_ref, o_ref):
    x = x_ref[...]
    wr = wr_ref[...]
    cents = c_ref[...]
    temp = t_ref[0, 0]

    z = jax.lax.dot_general(x, wr, (((1,), (1,)), ((), ())),
                            preferred_element_type=jnp.float32)
    z_norm = z / jnp.maximum(
        jnp.sqrt(jnp.sum(z * z, axis=-1, keepdims=True)), 1e-12)
    c_norm = cents / jnp.maximum(
        jnp.sqrt(jnp.sum(cents * cents, axis=-1, keepdims=True)), 1e-12)
    logits = jax.lax.dot_general(z_norm, c_norm, (((1,), (1,)), ((), ())),
                                 preferred_element_type=jnp.float32)
    logits = logits * jnp.exp(temp)
    probs = jax.nn.softmax(logits, axis=-1)

    i1 = jnp.argmax(probs, axis=-1)
    v1 = jnp.max(probs, axis=-1)
    head_iota = jax.lax.broadcasted_iota(jnp.int32, probs.shape, 1)
    masked = jnp.where(head_iota == i1[:, None], -jnp.inf, probs)
    i2 = jnp.argmax(masked, axis=-1)
    v2 = jnp.max(masked, axis=-1)
    s = v1 + v2 + 1e-6
    w1 = (v1 / s)[:, None]
    w2 = (v2 / s)[:, None]

    acc0 = jnp.zeros((x.shape[0], HEAD_DIM), dtype=jnp.float32)
    acc1 = jnp.zeros((x.shape[0], HEAD_DIM), dtype=jnp.float32)
    for h in range(NUM_HEADS):
        y_h = jnp.dot(x, w_ref[h], preferred_element_type=jnp.float32)
        y_h = y_h + b_ref[h][None, :]
        m0 = jnp.where(i1[:, None] == h, w1, 0.0)
        m1 = jnp.where(i2[:, None] == h, w2, 0.0)
        acc0 = acc0 + m0 * y_h
        acc1 = acc1 + m1 * y_h
    o_ref[:, :HEAD_DIM] = acc0
    o_ref[:, HEAD_DIM:] = acc1


@functools.partial(jax.jit, static_argnames=())
def kernel(x, Wr, centroids, temperature, weight, bias):
    batch_shape = x.shape[:-1]
    x_flat = x.reshape(-1, IN_FEATURES)
    T = x_flat.shape[0]
    out = pl.pallas_call(
        _sparse_query_kernel,
        out_shape=jax.ShapeDtypeStruct((T, TOP_K * HEAD_DIM), jnp.float32),
    )(x_flat, Wr, centroids, temperature.reshape(1, 1), weight, bias)
    return out.reshape(*batch_shape, TOP_K * HEAD_DIM)
